# Initial kernel scaffold; baseline (speedup 1.0000x reference)
import jax
import jax.numpy as jnp
from jax.experimental import pallas as pl


def kernel(s0, edge_index, matrix):
    raise NotImplementedError("write your pallas kernel here")


# speedup vs baseline: 58.3035x; 58.3035x over previous
import functools

import jax
import jax.numpy as jnp
from jax import lax
from jax.experimental import pallas as pl
from jax.experimental.pallas import tpu as pltpu
from jax.experimental.pallas import tpu_sc as plsc

ITERS = 20
N = 100000
S = 4
SW = 8
E = 6400000

NCORE = 2
NSUB = 16
WORKERS = NCORE * NSUB
NT = 3136
NPAD = NT * WORKERS
ROWS_PER_SUB = NPAD // NSUB

CHUNK = 128
EPT = 204800
NCH = EPT // CHUNK
NB = NCH // 2
EPAD = EPT * WORKERS

_mesh = plsc.VectorSubcoreMesh(core_axis_name="c", subcore_axis_name="s")
_params = pltpu.CompilerParams(use_tc_tiling_on_sc=False,
                               needs_layout_passes=False)


@functools.partial(
    pl.kernel,
    out_type=jax.ShapeDtypeStruct((NCORE, NPAD, SW), jnp.float32),
    mesh=_mesh,
    compiler_params=_params,
    scratch_types=[
        pltpu.VMEM((CHUNK,), jnp.int32),
        pltpu.VMEM((CHUNK,), jnp.int32),
        pltpu.VMEM((CHUNK, SW), jnp.float32),
        pltpu.VMEM((CHUNK,), jnp.int32),
        pltpu.VMEM((CHUNK,), jnp.int32),
        pltpu.VMEM((CHUNK, SW), jnp.float32),
        pltpu.VMEM_SHARED((NPAD, SW), jnp.float32),
        pltpu.VMEM_SHARED((NPAD, SW), jnp.float32),
        pltpu.SemaphoreType.DMA,
        pltpu.SemaphoreType.DMA,
        pltpu.SemaphoreType.DMA,
        pltpu.SemaphoreType.DMA,
    ],
)
def _scatter_phase(s_tab, srcs, dsts, zeros_in, out,
                   src0, dst0, msg0, src1, dst1, msg1,
                   acc, s_loc, isem0, isem1, gsem0, gsem1):
    cid = lax.axis_index("c")
    sid = lax.axis_index("s")
    wid = cid * NSUB + sid

    zbase = sid * ROWS_PER_SUB
    pltpu.sync_copy(zeros_in.at[pl.ds(zbase, ROWS_PER_SUB)],
                    acc.at[pl.ds(zbase, ROWS_PER_SUB)])
    pltpu.sync_copy(s_tab.at[pl.ds(zbase, ROWS_PER_SUB)],
                    s_loc.at[pl.ds(zbase, ROWS_PER_SUB)])
    plsc.subcore_barrier()

    pltpu.async_copy(srcs.at[wid, 0], src0, isem0)
    pltpu.async_copy(dsts.at[wid, 0], dst0, isem0)
    pltpu.async_copy(srcs.at[wid, 1], src1, isem1)
    pltpu.async_copy(dsts.at[wid, 1], dst1, isem1)

    def body(i, carry):
        pltpu.make_async_copy(srcs.at[wid, 0], src0, isem0).wait()
        pltpu.make_async_copy(dsts.at[wid, 0], dst0, isem0).wait()
        g0 = pltpu.async_copy(s_loc.at[src0], msg0, gsem0)
        pltpu.make_async_copy(srcs.at[wid, 1], src1, isem1).wait()
        pltpu.make_async_copy(dsts.at[wid, 1], dst1, isem1).wait()
        g1 = pltpu.async_copy(s_loc.at[src1], msg1, gsem1)
        g0.wait()
        pltpu.sync_copy(msg0, acc.at[dst0], add=True)

        @pl.when(i < NB - 1)
        def _():
            pltpu.async_copy(srcs.at[wid, 2 * i + 2], src0, isem0)
            pltpu.async_copy(dsts.at[wid, 2 * i + 2], dst0, isem0)

        g1.wait()
        pltpu.sync_copy(msg1, acc.at[dst1], add=True)

        @pl.when(i < NB - 1)
        def _():
            pltpu.async_copy(srcs.at[wid, 2 * i + 3], src1, isem1)
            pltpu.async_copy(dsts.at[wid, 2 * i + 3], dst1, isem1)

        return carry

    lax.fori_loop(0, NB, body, 0)
    plsc.subcore_barrier()

    pltpu.sync_copy(acc.at[pl.ds(zbase, ROWS_PER_SUB)],
                    out.at[cid, pl.ds(zbase, ROWS_PER_SUB)])


_FLOATS_PER_W = NT * SW
_GROUPS = NT // 16


@functools.partial(
    pl.kernel,
    out_type=jax.ShapeDtypeStruct((NPAD * SW,), jnp.float32),
    mesh=_mesh,
    compiler_params=_params,
    scratch_types=[
        pltpu.VMEM((_FLOATS_PER_W,), jnp.float32),
        pltpu.VMEM((_FLOATS_PER_W,), jnp.float32),
        pltpu.VMEM((_FLOATS_PER_W,), jnp.float32),
        pltpu.VMEM((2 ** S * S * S,), jnp.float32),
        pltpu.VMEM((_FLOATS_PER_W,), jnp.float32),
    ],
)
def _update_phase(s_flat, parts, t_flat, out, pa, pb, sv, tv, outv):
    cid = lax.axis_index("c")
    sid = lax.axis_index("s")
    wid = cid * NSUB + sid
    base = wid * _FLOATS_PER_W

    pltpu.sync_copy(parts.at[0, pl.ds(base, _FLOATS_PER_W)], pa)
    pltpu.sync_copy(parts.at[1, pl.ds(base, _FLOATS_PER_W)], pb)
    pltpu.sync_copy(s_flat.at[pl.ds(base, _FLOATS_PER_W)], sv)
    pltpu.sync_copy(t_flat, tv)

    lanes = lax.iota(jnp.int32, 16)
    zeros16 = jnp.zeros((16,), jnp.float32)

    def body(g, carry):
        b0 = g * (16 * SW) + lanes * SW
        xs = []
        num = None
        for s in range(S):
            tot = plsc.load_gather(pa, [b0 + s]) + plsc.load_gather(pb, [b0 + s])
            bit = jnp.where(tot > jnp.float32(0.5),
                            jnp.int32(1 << s), jnp.int32(0))
            num = bit if num is None else num + bit
            xs.append(plsc.load_gather(sv, [b0 + s]))
        tb = num * (S * S)
        for t in range(S):
            accv = xs[0] * plsc.load_gather(tv, [tb + t])
            for s in range(1, S):
                accv = accv + xs[s] * plsc.load_gather(tv, [tb + s * S + t])
            plsc.store_scatter(outv, [b0 + t], accv)
        for t in range(S, SW):
            plsc.store_scatter(outv, [b0 + t], zeros16)
        return carry

    lax.fori_loop(0, _GROUPS, body, 0)

    pltpu.sync_copy(outv, out.at[pl.ds(base, _FLOATS_PER_W)])


def kernel(s0, edge_index, matrix):
    s = jnp.zeros((NPAD, SW), jnp.float32).at[:N, :S].set(s0)
    src = edge_index[0]
    dst = edge_index[1]
    extra = EPAD - E
    pad_src = jnp.zeros((extra,), jnp.int32)
    pad_dst = N + (jnp.arange(extra, dtype=jnp.int32) % (NPAD - N))
    srcs = jnp.concatenate([src, pad_src]).reshape(WORKERS, NCH, CHUNK)
    dsts = jnp.concatenate([dst, pad_dst]).reshape(WORKERS, NCH, CHUNK)
    zeros_in = jnp.zeros((NPAD, SW), jnp.float32)
    t_flat = matrix.astype(jnp.float32).reshape(-1)

    for _ in range(ITERS):
        parts = _scatter_phase(s, srcs, dsts, zeros_in)
        s = _update_phase(s.reshape(-1), parts.reshape(NCORE, -1),
                          t_flat).reshape(NPAD, SW)
    return s[:N, :S]
